# baseline (speedup 1.0000x reference)
import functools

import jax
import jax.numpy as jnp
from jax import lax
from jax.experimental import pallas as pl
from jax.experimental.pallas import tpu as pltpu
from jax.experimental.pallas import tpu_sc as plsc

HIDDEN = 1024
NSL = HIDDEN // 16
EPS = 1e-05

NC = 2
NS = 16
NW = NC * NS

R = 32
RB = 8

_GATHER_DNUMS = lax.GatherDimensionNumbers(
    offset_dims=(), collapsed_slice_dims=(0,), start_index_map=(0,))


def _shuffle(v, idx):
    return lax.gather(v, idx[:, None], _GATHER_DNUMS, slice_sizes=(1,),
                      mode=lax.GatherScatterMode.PROMISE_IN_BOUNDS)


def _allsum(v):
    lanes = lax.iota(jnp.int32, 16)
    for k in (8, 4, 2, 1):
        v = v + _shuffle(v, (lanes + k) & 15)
    return v


def _rsqrt_vec(av):
    ii = lax.bitcast_convert_type(av, jnp.int32)
    ii = jnp.int32(0x5F3759DF) - (ii >> 1)
    y = lax.bitcast_convert_type(ii, jnp.float32)
    for _ in range(3):
        y = y * (1.5 - 0.5 * av * y * y)
    return y


def _make_sc_kernel(batch, seq_len):
    n_rows = batch * seq_len
    s_per_w = seq_len // NW
    nsub = s_per_w // R
    mesh = plsc.VectorSubcoreMesh(core_axis_name="c", subcore_axis_name="s")

    @functools.partial(
        pl.kernel,
        out_type=jax.ShapeDtypeStruct((n_rows, HIDDEN), jnp.float32),
        mesh=mesh,
        scratch_types=[
            pltpu.VMEM((batch, s_per_w), jnp.int32),
            pltpu.VMEM((R, HIDDEN), jnp.float32),
            pltpu.VMEM((R, HIDDEN), jnp.float32),
            pltpu.VMEM((R, HIDDEN), jnp.float32),
            pltpu.VMEM((HIDDEN,), jnp.float32),
            pltpu.VMEM((HIDDEN,), jnp.float32),
            pltpu.VMEM((HIDDEN,), jnp.float32),
            pltpu.SemaphoreType.DMA,
            pltpu.SemaphoreType.DMA,
            pltpu.SemaphoreType.DMA,
            pltpu.SemaphoreType.DMA,
        ],
    )
    def sc_kernel(x_hbm, wemb_hbm, pos_hbm, type_hbm, gamma_hbm, beta_hbm,
                  out_hbm, idx_all, w0, w1, posv, tv, gv, bv,
                  sg0, sg1, so0, so1):
        wid = lax.axis_index("s") * NC + lax.axis_index("c")
        s_base = wid * s_per_w

        for bb in range(batch):
            pltpu.sync_copy(x_hbm.at[bb, pl.ds(s_base, s_per_w)],
                            idx_all.at[bb])
        pltpu.sync_copy(type_hbm, tv)
        pltpu.sync_copy(gamma_hbm, gv)
        pltpu.sync_copy(beta_hbm, bv)

        wvs = (w0, w1)
        sgs = (sg0, sg1)
        sos = (so0, so1)
        iters = [(sc_i, b) for sc_i in range(nsub) for b in range(batch)]
        nit = len(iters)

        def start_gather(i):
            sc_i, b = iters[i]
            buf = i % 2
            return pltpu.async_copy(
                wemb_hbm.at[idx_all.at[b, pl.ds(sc_i * R, R)]], wvs[buf],
                sgs[buf])

        def normalize(wv):
            def rblk_body(rb, _):
                r0 = rb * RB

                zeros = tuple(jnp.zeros((16,), jnp.float32)
                              for _ in range(2 * RB))

                @plsc.parallel_loop(0, NSL, unroll=4, carry=zeros)
                def acc(j, acc_in):
                    sl = pl.ds(j * 16, 16)
                    tvj = tv[sl]
                    es = []
                    for k in range(RB):
                        e = wv[r0 + k, sl] + posv[r0 + k, sl] + tvj
                        wv[r0 + k, sl] = e
                        es.append(e)
                    return (tuple(acc_in[k] + es[k] for k in range(RB)) +
                            tuple(acc_in[RB + k] + es[k] * es[k]
                                  for k in range(RB)))

                ms, ys = [], []
                for k in range(RB):
                    m = _allsum(acc[k]) * (1.0 / HIDDEN)
                    q = _allsum(acc[RB + k]) * (1.0 / HIDDEN)
                    ms.append(m)
                    ys.append(_rsqrt_vec(q - m * m + EPS))

                @plsc.parallel_loop(0, NSL, unroll=4)
                def _p2(j):
                    sl = pl.ds(j * 16, 16)
                    gj = gv[sl]
                    bj = bv[sl]
                    for k in range(RB):
                        a = ys[k] * gj
                        s = bj - ms[k] * a
                        wv[r0 + k, sl] = wv[r0 + k, sl] * a + s

                return _

            lax.fori_loop(0, R // RB, rblk_body, None)

        g_desc = [None] * nit
        o_desc = [None] * nit
        g_desc[0] = start_gather(0)
        for i in range(nit):
            sc_i, b = iters[i]
            buf = i % 2
            if b == 0:
                pltpu.sync_copy(pos_hbm.at[pl.ds(s_base + sc_i * R, R)],
                                posv)
            g_desc[i].wait()
            if i + 1 < nit:
                if i >= 1:
                    o_desc[i - 1].wait()
                g_desc[i + 1] = start_gather(i + 1)
            normalize(wvs[buf])
            s0 = s_base + sc_i * R
            o_desc[i] = pltpu.async_copy(
                wvs[buf], out_hbm.at[pl.ds(b * seq_len + s0, R)], sos[buf])
        o_desc[nit - 2].wait()
        o_desc[nit - 1].wait()

    return sc_kernel


def kernel(x, word_emb, pos_emb, type_emb, gamma, beta):
    b, s = x.shape
    sc = _make_sc_kernel(b, s)
    out_flat = sc(x.astype(jnp.int32), word_emb, pos_emb[:s],
                  type_emb.reshape(-1), gamma, beta)
    return out_flat.reshape(b, s, HIDDEN)


# speedup vs baseline: 1.0434x; 1.0434x over previous
import functools

import jax
import jax.numpy as jnp
from jax import lax
from jax.experimental import pallas as pl
from jax.experimental.pallas import tpu as pltpu
from jax.experimental.pallas import tpu_sc as plsc

HIDDEN = 1024
NSL = HIDDEN // 16
EPS = 1e-05

NC = 2
NS = 16
NW = NC * NS

R = 32
RB = 8

_GATHER_DNUMS = lax.GatherDimensionNumbers(
    offset_dims=(), collapsed_slice_dims=(0,), start_index_map=(0,))


def _shuffle(v, idx):
    return lax.gather(v, idx[:, None], _GATHER_DNUMS, slice_sizes=(1,),
                      mode=lax.GatherScatterMode.PROMISE_IN_BOUNDS)


def _allsum(v):
    lanes = lax.iota(jnp.int32, 16)
    for k in (8, 4, 2, 1):
        v = v + _shuffle(v, (lanes + k) & 15)
    return v


def _rsqrt_vec(av):
    ii = lax.bitcast_convert_type(av, jnp.int32)
    ii = jnp.int32(0x5F3759DF) - (ii >> 1)
    y = lax.bitcast_convert_type(ii, jnp.float32)
    for _ in range(3):
        y = y * (1.5 - 0.5 * av * y * y)
    return y


def _make_sc_kernel(batch, seq_len):
    n_rows = batch * seq_len
    s_per_w = seq_len // NW
    nsub = s_per_w // R
    mesh = plsc.VectorSubcoreMesh(core_axis_name="c", subcore_axis_name="s")

    @functools.partial(
        pl.kernel,
        out_type=jax.ShapeDtypeStruct((n_rows, HIDDEN), jnp.float32),
        mesh=mesh,
        scratch_types=[
            pltpu.VMEM((batch, s_per_w), jnp.int32),
            pltpu.VMEM((R, HIDDEN), jnp.float32),
            pltpu.VMEM((R, HIDDEN), jnp.float32),
            pltpu.VMEM((R, HIDDEN), jnp.float32),
            pltpu.VMEM((HIDDEN,), jnp.float32),
            pltpu.VMEM((HIDDEN,), jnp.float32),
            pltpu.VMEM((HIDDEN,), jnp.float32),
            pltpu.SemaphoreType.DMA,
            pltpu.SemaphoreType.DMA,
            pltpu.SemaphoreType.DMA,
            pltpu.SemaphoreType.DMA,
        ],
    )
    def sc_kernel(x_hbm, wemb_hbm, pos_hbm, type_hbm, gamma_hbm, beta_hbm,
                  out_hbm, idx_all, w0, w1, posv, tv, gv, bv,
                  sg0, sg1, so0, so1):
        wid = lax.axis_index("s") * NC + lax.axis_index("c")
        s_base = wid * s_per_w

        for bb in range(batch):
            pltpu.sync_copy(x_hbm.at[bb, pl.ds(s_base, s_per_w)],
                            idx_all.at[bb])
        pltpu.sync_copy(type_hbm, tv)
        pltpu.sync_copy(gamma_hbm, gv)
        pltpu.sync_copy(beta_hbm, bv)

        wvs = (w0, w1)
        sgs = (sg0, sg1)
        sos = (so0, so1)
        iters = [(sc_i, b) for sc_i in range(nsub) for b in range(batch)]
        nit = len(iters)

        def start_gather(i):
            sc_i, b = iters[i]
            buf = i % 2
            return pltpu.async_copy(
                wemb_hbm.at[idx_all.at[b, pl.ds(sc_i * R, R)]], wvs[buf],
                sgs[buf])

        def normalize(wv):
            def rblk_body(rb, _):
                r0 = rb * RB

                zeros = tuple(jnp.zeros((16,), jnp.float32)
                              for _ in range(2 * RB))

                @plsc.parallel_loop(0, NSL, unroll=2, carry=zeros)
                def acc(j, acc_in):
                    sl = pl.ds(j * 16, 16)
                    tvj = tv[sl]
                    es = []
                    for k in range(RB):
                        e = wv[r0 + k, sl] + posv[r0 + k, sl] + tvj
                        wv[r0 + k, sl] = e
                        es.append(e)
                    return (tuple(acc_in[k] + es[k] for k in range(RB)) +
                            tuple(acc_in[RB + k] + es[k] * es[k]
                                  for k in range(RB)))

                ms, ys = [], []
                for k in range(RB):
                    m = _allsum(acc[k]) * (1.0 / HIDDEN)
                    q = _allsum(acc[RB + k]) * (1.0 / HIDDEN)
                    ms.append(m)
                    ys.append(_rsqrt_vec(q - m * m + EPS))

                @plsc.parallel_loop(0, NSL, unroll=2)
                def _p2(j):
                    sl = pl.ds(j * 16, 16)
                    gj = gv[sl]
                    bj = bv[sl]
                    for k in range(RB):
                        a = ys[k] * gj
                        s = bj - ms[k] * a
                        wv[r0 + k, sl] = wv[r0 + k, sl] * a + s

                return _

            lax.fori_loop(0, R // RB, rblk_body, None)

        g_desc = [None] * nit
        o_desc = [None] * nit
        g_desc[0] = start_gather(0)
        for i in range(nit):
            sc_i, b = iters[i]
            buf = i % 2
            if b == 0:
                pltpu.sync_copy(pos_hbm.at[pl.ds(s_base + sc_i * R, R)],
                                posv)
            g_desc[i].wait()
            if i + 1 < nit:
                if i >= 1:
                    o_desc[i - 1].wait()
                g_desc[i + 1] = start_gather(i + 1)
            normalize(wvs[buf])
            s0 = s_base + sc_i * R
            o_desc[i] = pltpu.async_copy(
                wvs[buf], out_hbm.at[pl.ds(b * seq_len + s0, R)], sos[buf])
        o_desc[nit - 2].wait()
        o_desc[nit - 1].wait()

    return sc_kernel


def kernel(x, word_emb, pos_emb, type_emb, gamma, beta):
    b, s = x.shape
    sc = _make_sc_kernel(b, s)
    out_flat = sc(x.astype(jnp.int32), word_emb, pos_emb[:s],
                  type_emb.reshape(-1), gamma, beta)
    return out_flat.reshape(b, s, HIDDEN)


# speedup vs baseline: 1.0942x; 1.0486x over previous
import functools

import jax
import jax.numpy as jnp
from jax import lax
from jax.experimental import pallas as pl
from jax.experimental.pallas import tpu as pltpu
from jax.experimental.pallas import tpu_sc as plsc

HIDDEN = 1024
NSL = HIDDEN // 16
EPS = 1e-05

NC = 2
NS = 16
NW = NC * NS

R = 32
RB = 8

_GATHER_DNUMS = lax.GatherDimensionNumbers(
    offset_dims=(), collapsed_slice_dims=(0,), start_index_map=(0,))


def _shuffle(v, idx):
    return lax.gather(v, idx[:, None], _GATHER_DNUMS, slice_sizes=(1,),
                      mode=lax.GatherScatterMode.PROMISE_IN_BOUNDS)


def _allsum(v):
    lanes = lax.iota(jnp.int32, 16)
    for k in (8, 4, 2, 1):
        v = v + _shuffle(v, (lanes + k) & 15)
    return v


def _rsqrt_vec(av):
    ii = lax.bitcast_convert_type(av, jnp.int32)
    ii = jnp.int32(0x5F3759DF) - (ii >> 1)
    y = lax.bitcast_convert_type(ii, jnp.float32)
    for _ in range(3):
        y = y * (1.5 - 0.5 * av * y * y)
    return y


def _make_sc_kernel(batch, seq_len):
    n_rows = batch * seq_len
    s_per_w = seq_len // NW
    nsub = s_per_w // R
    mesh = plsc.VectorSubcoreMesh(core_axis_name="c", subcore_axis_name="s")

    @functools.partial(
        pl.kernel,
        out_type=jax.ShapeDtypeStruct((n_rows, HIDDEN), jnp.float32),
        mesh=mesh,
        scratch_types=[
            pltpu.VMEM((batch, s_per_w), jnp.int32),
            pltpu.VMEM((R, HIDDEN), jnp.float32),
            pltpu.VMEM((R, HIDDEN), jnp.float32),
            pltpu.VMEM((R, HIDDEN), jnp.float32),
            pltpu.VMEM((HIDDEN,), jnp.float32),
            pltpu.VMEM((HIDDEN,), jnp.float32),
            pltpu.VMEM((HIDDEN,), jnp.float32),
            pltpu.SemaphoreType.DMA,
            pltpu.SemaphoreType.DMA,
            pltpu.SemaphoreType.DMA,
            pltpu.SemaphoreType.DMA,
            pltpu.SemaphoreType.DMA,
            pltpu.SemaphoreType.DMA,
        ],
    )
    def sc_kernel(x_hbm, wemb_hbm, pos_hbm, type_hbm, gamma_hbm, beta_hbm,
                  out_hbm, idx_all, w0, w1, posv, tv, gv, bv,
                  sg0, sg1, so0, so1, sx, sp):
        wid = lax.axis_index("s") * NC + lax.axis_index("c")
        s_base = wid * s_per_w

        pltpu.sync_copy(x_hbm.at[0, pl.ds(s_base, s_per_w)], idx_all.at[0])
        aux = [pltpu.async_copy(x_hbm.at[bb, pl.ds(s_base, s_per_w)],
                                idx_all.at[bb], sx)
               for bb in range(1, batch)]
        aux.append(pltpu.async_copy(type_hbm, tv, sx))
        aux.append(pltpu.async_copy(gamma_hbm, gv, sx))
        aux.append(pltpu.async_copy(beta_hbm, bv, sx))

        wvs = (w0, w1)
        sgs = (sg0, sg1)
        sos = (so0, so1)
        iters = [(sc_i, b) for sc_i in range(nsub) for b in range(batch)]
        nit = len(iters)

        def start_gather(i):
            sc_i, b = iters[i]
            buf = i % 2
            return pltpu.async_copy(
                wemb_hbm.at[idx_all.at[b, pl.ds(sc_i * R, R)]], wvs[buf],
                sgs[buf])

        def normalize(wv):
            def rblk_body(rb, _):
                r0 = rb * RB

                zeros = tuple(jnp.zeros((16,), jnp.float32)
                              for _ in range(2 * RB))

                @plsc.parallel_loop(0, NSL, unroll=1, carry=zeros)
                def acc(j, acc_in):
                    sl = pl.ds(j * 16, 16)
                    es = []
                    for k in range(RB):
                        e = wv[r0 + k, sl] + posv[r0 + k, sl]
                        wv[r0 + k, sl] = e
                        es.append(e)
                    return (tuple(acc_in[k] + es[k] for k in range(RB)) +
                            tuple(acc_in[RB + k] + es[k] * es[k]
                                  for k in range(RB)))

                ms, ys = [], []
                for k in range(RB):
                    m = _allsum(acc[k]) * (1.0 / HIDDEN)
                    q = _allsum(acc[RB + k]) * (1.0 / HIDDEN)
                    ms.append(m)
                    ys.append(_rsqrt_vec(q - m * m + EPS))

                @plsc.parallel_loop(0, NSL, unroll=2)
                def _p2(j):
                    sl = pl.ds(j * 16, 16)
                    gj = gv[sl]
                    bj = bv[sl]
                    for k in range(RB):
                        a = ys[k] * gj
                        s = bj - ms[k] * a
                        wv[r0 + k, sl] = wv[r0 + k, sl] * a + s

                return _

            lax.fori_loop(0, R // RB, rblk_body, None)

        def tvadd():
            @plsc.parallel_loop(0, NSL)
            def _tva(j):
                sl = pl.ds(j * 16, 16)
                tvj = tv[sl]
                for r in range(R):
                    posv[r, sl] = posv[r, sl] + tvj

        g_desc = [None] * nit
        o_desc = [None] * nit
        g_desc[0] = start_gather(0)
        pltpu.sync_copy(pos_hbm.at[pl.ds(s_base, R)], posv)
        for d in aux:
            d.wait()
        p_desc = None
        for i in range(nit):
            sc_i, b = iters[i]
            buf = i % 2
            if b == 0:
                if p_desc is not None:
                    p_desc.wait()
                tvadd()
            g_desc[i].wait()
            if i + 1 < nit:
                if i >= 1:
                    o_desc[i - 1].wait()
                g_desc[i + 1] = start_gather(i + 1)
            normalize(wvs[buf])
            if b == batch - 1 and sc_i + 1 < nsub:
                p_desc = pltpu.async_copy(
                    pos_hbm.at[pl.ds(s_base + (sc_i + 1) * R, R)], posv, sp)
            s0 = s_base + sc_i * R
            o_desc[i] = pltpu.async_copy(
                wvs[buf], out_hbm.at[pl.ds(b * seq_len + s0, R)], sos[buf])
        o_desc[nit - 2].wait()
        o_desc[nit - 1].wait()

    return sc_kernel


def kernel(x, word_emb, pos_emb, type_emb, gamma, beta):
    b, s = x.shape
    sc = _make_sc_kernel(b, s)
    out_flat = sc(x.astype(jnp.int32), word_emb, pos_emb[:s],
                  type_emb.reshape(-1), gamma, beta)
    return out_flat.reshape(b, s, HIDDEN)


# speedup vs baseline: 1.3323x; 1.2177x over previous
import functools

import jax
import jax.numpy as jnp
from jax import lax
from jax.experimental import pallas as pl
from jax.experimental.pallas import tpu as pltpu
from jax.experimental.pallas import tpu_sc as plsc

HIDDEN = 1024
NSL = HIDDEN // 16
EPS = 1e-05

NC = 2
NS = 16
NW = NC * NS

R = 32
RB = 4

_GATHER_DNUMS = lax.GatherDimensionNumbers(
    offset_dims=(), collapsed_slice_dims=(0,), start_index_map=(0,))


def _shuffle(v, idx):
    return lax.gather(v, idx[:, None], _GATHER_DNUMS, slice_sizes=(1,),
                      mode=lax.GatherScatterMode.PROMISE_IN_BOUNDS)


def _allsum(v):
    lanes = lax.iota(jnp.int32, 16)
    for k in (8, 4, 2, 1):
        v = v + _shuffle(v, (lanes + k) & 15)
    return v


def _rsqrt_vec(av):
    ii = lax.bitcast_convert_type(av, jnp.int32)
    ii = jnp.int32(0x5F3759DF) - (ii >> 1)
    y = lax.bitcast_convert_type(ii, jnp.float32)
    for _ in range(3):
        y = y * (1.5 - 0.5 * av * y * y)
    return y


def _make_sc_kernel(batch, seq_len):
    n_rows = batch * seq_len
    s_per_w = seq_len // NW
    nsub = s_per_w // R
    mesh = plsc.VectorSubcoreMesh(core_axis_name="c", subcore_axis_name="s")

    @functools.partial(
        pl.kernel,
        out_type=jax.ShapeDtypeStruct((n_rows, HIDDEN), jnp.float32),
        mesh=mesh,
        scratch_types=[
            pltpu.VMEM((batch, s_per_w), jnp.int32),
            pltpu.VMEM((R, HIDDEN), jnp.float32),
            pltpu.VMEM((R, HIDDEN), jnp.float32),
            pltpu.VMEM((R, HIDDEN), jnp.float32),
            pltpu.VMEM((HIDDEN,), jnp.float32),
            pltpu.VMEM((HIDDEN,), jnp.float32),
            pltpu.VMEM((HIDDEN,), jnp.float32),
            pltpu.SemaphoreType.DMA,
            pltpu.SemaphoreType.DMA,
            pltpu.SemaphoreType.DMA,
            pltpu.SemaphoreType.DMA,
            pltpu.SemaphoreType.DMA,
            pltpu.SemaphoreType.DMA,
        ],
    )
    def sc_kernel(x_hbm, wemb_hbm, pos_hbm, type_hbm, gamma_hbm, beta_hbm,
                  out_hbm, idx_all, w0, w1, posv, tv, gv, bv,
                  sg0, sg1, so0, so1, sx, sp):
        wid = lax.axis_index("s") * NC + lax.axis_index("c")
        s_base = wid * s_per_w

        pltpu.sync_copy(x_hbm.at[0, pl.ds(s_base, s_per_w)], idx_all.at[0])
        aux = [pltpu.async_copy(x_hbm.at[bb, pl.ds(s_base, s_per_w)],
                                idx_all.at[bb], sx)
               for bb in range(1, batch)]
        aux.append(pltpu.async_copy(type_hbm, tv, sx))
        aux.append(pltpu.async_copy(gamma_hbm, gv, sx))
        aux.append(pltpu.async_copy(beta_hbm, bv, sx))

        wvs = (w0, w1)
        sgs = (sg0, sg1)
        sos = (so0, so1)
        iters = [(sc_i, b) for sc_i in range(nsub) for b in range(batch)]
        nit = len(iters)

        def start_gather(i):
            sc_i, b = iters[i]
            buf = i % 2
            return pltpu.async_copy(
                wemb_hbm.at[idx_all.at[b, pl.ds(sc_i * R, R)]], wvs[buf],
                sgs[buf])

        def normalize(wv):
            def rblk_body(rb, _):
                r0 = rb * RB

                zeros = tuple(jnp.zeros((16,), jnp.float32)
                              for _ in range(2 * RB))

                @plsc.parallel_loop(0, NSL, unroll=2, carry=zeros)
                def acc(j, acc_in):
                    sl = pl.ds(j * 16, 16)
                    es = []
                    for k in range(RB):
                        e = wv[r0 + k, sl] + posv[r0 + k, sl]
                        wv[r0 + k, sl] = e
                        es.append(e)
                    return (tuple(acc_in[k] + es[k] for k in range(RB)) +
                            tuple(acc_in[RB + k] + es[k] * es[k]
                                  for k in range(RB)))

                ms, ys = [], []
                for k in range(RB):
                    m = _allsum(acc[k]) * (1.0 / HIDDEN)
                    q = _allsum(acc[RB + k]) * (1.0 / HIDDEN)
                    ms.append(m)
                    ys.append(_rsqrt_vec(q - m * m + EPS))

                @plsc.parallel_loop(0, NSL, unroll=2)
                def _p2(j):
                    sl = pl.ds(j * 16, 16)
                    gj = gv[sl]
                    bj = bv[sl]
                    for k in range(RB):
                        a = ys[k] * gj
                        s = bj - ms[k] * a
                        wv[r0 + k, sl] = wv[r0 + k, sl] * a + s

                return _

            lax.fori_loop(0, R // RB, rblk_body, None)

        def tvadd():
            @plsc.parallel_loop(0, NSL)
            def _tva(j):
                sl = pl.ds(j * 16, 16)
                tvj = tv[sl]
                for r in range(R):
                    posv[r, sl] = posv[r, sl] + tvj

        g_desc = [None] * nit
        o_desc = [None] * nit
        g_desc[0] = start_gather(0)
        pltpu.sync_copy(pos_hbm.at[pl.ds(s_base, R)], posv)
        for d in aux:
            d.wait()
        p_desc = None
        for i in range(nit):
            sc_i, b = iters[i]
            buf = i % 2
            if b == 0:
                if p_desc is not None:
                    p_desc.wait()
                tvadd()
            g_desc[i].wait()
            if i + 1 < nit:
                if i >= 1:
                    o_desc[i - 1].wait()
                g_desc[i + 1] = start_gather(i + 1)
            normalize(wvs[buf])
            if b == batch - 1 and sc_i + 1 < nsub:
                p_desc = pltpu.async_copy(
                    pos_hbm.at[pl.ds(s_base + (sc_i + 1) * R, R)], posv, sp)
            s0 = s_base + sc_i * R
            o_desc[i] = pltpu.async_copy(
                wvs[buf], out_hbm.at[pl.ds(b * seq_len + s0, R)], sos[buf])
        o_desc[nit - 2].wait()
        o_desc[nit - 1].wait()

    return sc_kernel


def kernel(x, word_emb, pos_emb, type_emb, gamma, beta):
    b, s = x.shape
    sc = _make_sc_kernel(b, s)
    out_flat = sc(x.astype(jnp.int32), word_emb, pos_emb[:s],
                  type_emb.reshape(-1), gamma, beta)
    return out_flat.reshape(b, s, HIDDEN)


# speedup vs baseline: 1.3437x; 1.0086x over previous
import functools

import jax
import jax.numpy as jnp
from jax import lax
from jax.experimental import pallas as pl
from jax.experimental.pallas import tpu as pltpu
from jax.experimental.pallas import tpu_sc as plsc

HIDDEN = 1024
NSL = HIDDEN // 16
EPS = 1e-05

NC = 2
NS = 16
NW = NC * NS

R = 32
RB = 4

_GATHER_DNUMS = lax.GatherDimensionNumbers(
    offset_dims=(), collapsed_slice_dims=(0,), start_index_map=(0,))


def _shuffle(v, idx):
    return lax.gather(v, idx[:, None], _GATHER_DNUMS, slice_sizes=(1,),
                      mode=lax.GatherScatterMode.PROMISE_IN_BOUNDS)


def _allsum(v):
    lanes = lax.iota(jnp.int32, 16)
    for k in (8, 4, 2, 1):
        v = v + _shuffle(v, (lanes + k) & 15)
    return v


def _rsqrt_vec(av):
    ii = lax.bitcast_convert_type(av, jnp.int32)
    ii = jnp.int32(0x5F3759DF) - (ii >> 1)
    y = lax.bitcast_convert_type(ii, jnp.float32)
    for _ in range(2):
        y = y * (1.5 - 0.5 * av * y * y)
    return y


def _make_sc_kernel(batch, seq_len):
    n_rows = batch * seq_len
    s_per_w = seq_len // NW
    nsub = s_per_w // R
    mesh = plsc.VectorSubcoreMesh(core_axis_name="c", subcore_axis_name="s")

    @functools.partial(
        pl.kernel,
        out_type=jax.ShapeDtypeStruct((n_rows, HIDDEN), jnp.float32),
        mesh=mesh,
        scratch_types=[
            pltpu.VMEM((batch, s_per_w), jnp.int32),
            pltpu.VMEM((R, HIDDEN), jnp.float32),
            pltpu.VMEM((R, HIDDEN), jnp.float32),
            pltpu.VMEM((R, HIDDEN), jnp.float32),
            pltpu.VMEM((HIDDEN,), jnp.float32),
            pltpu.VMEM((HIDDEN,), jnp.float32),
            pltpu.VMEM((HIDDEN,), jnp.float32),
            pltpu.SemaphoreType.DMA,
            pltpu.SemaphoreType.DMA,
            pltpu.SemaphoreType.DMA,
            pltpu.SemaphoreType.DMA,
            pltpu.SemaphoreType.DMA,
            pltpu.SemaphoreType.DMA,
        ],
    )
    def sc_kernel(x_hbm, wemb_hbm, pos_hbm, type_hbm, gamma_hbm, beta_hbm,
                  out_hbm, idx_all, w0, w1, posv, tv, gv, bv,
                  sg0, sg1, so0, so1, sx, sp):
        wid = lax.axis_index("s") * NC + lax.axis_index("c")
        s_base = wid * s_per_w

        pltpu.sync_copy(x_hbm.at[0, pl.ds(s_base, s_per_w)], idx_all.at[0])
        aux = [pltpu.async_copy(x_hbm.at[bb, pl.ds(s_base, s_per_w)],
                                idx_all.at[bb], sx)
               for bb in range(1, batch)]
        aux.append(pltpu.async_copy(type_hbm, tv, sx))
        aux.append(pltpu.async_copy(gamma_hbm, gv, sx))
        aux.append(pltpu.async_copy(beta_hbm, bv, sx))

        wvs = (w0, w1)
        sgs = (sg0, sg1)
        sos = (so0, so1)
        iters = [(sc_i, b) for sc_i in range(nsub) for b in range(batch)]
        nit = len(iters)

        def start_gather(i):
            sc_i, b = iters[i]
            buf = i % 2
            return pltpu.async_copy(
                wemb_hbm.at[idx_all.at[b, pl.ds(sc_i * R, R)]], wvs[buf],
                sgs[buf])

        def normalize(wv):
            def rblk_body(rb, _):
                r0 = rb * RB

                zeros = tuple(jnp.zeros((16,), jnp.float32)
                              for _ in range(2 * RB))

                @plsc.parallel_loop(0, NSL, unroll=2, carry=zeros)
                def acc(j, acc_in):
                    sl = pl.ds(j * 16, 16)
                    es = []
                    for k in range(RB):
                        e = wv[r0 + k, sl] + posv[r0 + k, sl]
                        wv[r0 + k, sl] = e
                        es.append(e)
                    return (tuple(acc_in[k] + es[k] for k in range(RB)) +
                            tuple(acc_in[RB + k] + es[k] * es[k]
                                  for k in range(RB)))

                ms, ys = [], []
                for k in range(RB):
                    m = _allsum(acc[k]) * (1.0 / HIDDEN)
                    q = _allsum(acc[RB + k]) * (1.0 / HIDDEN)
                    ms.append(m)
                    ys.append(_rsqrt_vec(q - m * m + EPS))

                @plsc.parallel_loop(0, NSL, unroll=2)
                def _p2(j):
                    sl = pl.ds(j * 16, 16)
                    gj = gv[sl]
                    bj = bv[sl]
                    for k in range(RB):
                        a = ys[k] * gj
                        s = bj - ms[k] * a
                        wv[r0 + k, sl] = wv[r0 + k, sl] * a + s

                return _

            lax.fori_loop(0, R // RB, rblk_body, None)

        def tvadd():
            @plsc.parallel_loop(0, NSL)
            def _tva(j):
                sl = pl.ds(j * 16, 16)
                tvj = tv[sl]
                for r in range(R):
                    posv[r, sl] = posv[r, sl] + tvj

        g_desc = [None] * nit
        o_desc = [None] * nit
        g_desc[0] = start_gather(0)
        pltpu.sync_copy(pos_hbm.at[pl.ds(s_base, R)], posv)
        for d in aux:
            d.wait()
        p_desc = None
        for i in range(nit):
            sc_i, b = iters[i]
            buf = i % 2
            if b == 0:
                if p_desc is not None:
                    p_desc.wait()
                tvadd()
            g_desc[i].wait()
            if i + 1 < nit:
                if i >= 1:
                    o_desc[i - 1].wait()
                g_desc[i + 1] = start_gather(i + 1)
            normalize(wvs[buf])
            if b == batch - 1 and sc_i + 1 < nsub:
                p_desc = pltpu.async_copy(
                    pos_hbm.at[pl.ds(s_base + (sc_i + 1) * R, R)], posv, sp)
            s0 = s_base + sc_i * R
            o_desc[i] = pltpu.async_copy(
                wvs[buf], out_hbm.at[pl.ds(b * seq_len + s0, R)], sos[buf])
        o_desc[nit - 2].wait()
        o_desc[nit - 1].wait()

    return sc_kernel


def kernel(x, word_emb, pos_emb, type_emb, gamma, beta):
    b, s = x.shape
    sc = _make_sc_kernel(b, s)
    out_flat = sc(x.astype(jnp.int32), word_emb, pos_emb[:s],
                  type_emb.reshape(-1), gamma, beta)
    return out_flat.reshape(b, s, HIDDEN)
